# baseline (speedup 1.0000x reference)
import functools

import jax
import jax.numpy as jnp
from jax import lax
from jax.experimental import pallas as pl
from jax.experimental.pallas import tpu as pltpu
from jax.experimental.pallas import tpu_sc as plsc

N = 10000
NP = 10240
E = 160000
CHUNK = 128
CCNT = 128
NSUB = 16
CPS = 80
EPSC = NSUB * CPS * CHUNK
RPS = NP // NSUB
BM = 512
NBLK = NP // BM


def _make_sc_segsum():
    scratch = [
        pltpu.VMEM((1, CHUNK), jnp.int32),
        pltpu.VMEM((1, CHUNK), jnp.int32),
        pltpu.VMEM((CHUNK, 128), jnp.float32),
        pltpu.VMEM_SHARED((NP, 128), jnp.float32),
        pltpu.SemaphoreType.DMA,
    ]
    mesh = plsc.VectorSubcoreMesh(core_axis_name="c", subcore_axis_name="s")

    def body(src2, dst2, h, zrows, agg_out, sv, dv, rows, acc, sem):
        c = lax.axis_index("c")
        s = lax.axis_index("s")
        base = s * RPS
        row0 = (c * NSUB + s) * CPS
        pltpu.sync_copy(zrows.at[pl.ds(0, CHUNK)], rows)
        for k in range(RPS // CHUNK):
            pltpu.sync_copy(rows, acc.at[pl.ds(base + k * CHUNK, CHUNK)])
        plsc.subcore_barrier()

        def step(q, carry):
            r = row0 + q
            pltpu.sync_copy(src2.at[r], sv)
            pltpu.sync_copy(dst2.at[r], dv)
            pltpu.async_copy(h.at[sv.at[0]], rows, sem).wait()
            pltpu.sync_copy(rows, acc.at[dv.at[0]], add=True)
            return carry

        lax.fori_loop(0, CPS, step, 0)
        plsc.subcore_barrier()

        for k in range(RPS // CHUNK):
            pltpu.sync_copy(acc.at[pl.ds(base + k * CHUNK, CHUNK)], rows)
            pltpu.sync_copy(rows, agg_out.at[pl.ds(c * NP + base + k * CHUNK, CHUNK)])

    return functools.partial(
        pl.kernel, mesh=mesh,
        out_type=jax.ShapeDtypeStruct((2 * NP, 128), jnp.float32),
        scratch_types=scratch,
    )(body)


CCPS = EPSC // CCNT // NSUB


def _make_sc_count():
    scratch = [
        pltpu.VMEM((CCPS, 1, CCNT), jnp.int32),
        pltpu.VMEM((CCNT, 128), jnp.float32),
        pltpu.VMEM_SHARED((NP, 128), jnp.float32),
        pltpu.SemaphoreType.DMA,
    ]
    mesh = plsc.VectorSubcoreMesh(core_axis_name="c", subcore_axis_name="s")

    def body(dst2, zrows, ones, cnt_out, dv, rows, acc, sem_s):
        c = lax.axis_index("c")
        s = lax.axis_index("s")
        base = s * RPS
        pltpu.sync_copy(dst2.at[pl.ds((c * NSUB + s) * CCPS, CCPS)], dv)
        pltpu.sync_copy(zrows.at[pl.ds(0, CCNT)], rows)
        for k in range(RPS // CCNT):
            pltpu.sync_copy(rows, acc.at[pl.ds(base + k * CCNT, CCNT)])
        pltpu.sync_copy(ones, rows)
        plsc.subcore_barrier()

        FIRE = 8

        def step(t, carry):
            for j in range(FIRE):
                pltpu.make_async_copy(rows, acc.at[dv.at[FIRE * t + j, 0]],
                                      sem_s).start(add=True)
            for j in range(FIRE):
                pltpu.make_async_copy(rows, acc.at[pl.ds(0, CCNT)], sem_s).wait()
            return carry

        lax.fori_loop(0, CCPS // FIRE, step, 0)
        plsc.subcore_barrier()

        for k in range(RPS // CCNT):
            pltpu.sync_copy(acc.at[pl.ds(base + k * CCNT, CCNT)], rows)
            pltpu.sync_copy(rows, cnt_out.at[pl.ds(c * NP + base + k * CCNT, CCNT)])

    return functools.partial(
        pl.kernel, mesh=mesh,
        out_type=jax.ShapeDtypeStruct((2 * NP, 128), jnp.float32),
        scratch_types=scratch,
    )(body)


@functools.lru_cache(maxsize=None)
def _sc_segsum():
    return _make_sc_segsum()


@functools.lru_cache(maxsize=None)
def _sc_count():
    return _make_sc_count()


def _tc_first_body(x_ref, w_ref, b_ref, h_ref, z_ref, *, O):
    x = x_ref[...]
    y = jnp.dot(x, w_ref[0], preferred_element_type=jnp.float32)
    h_ref[...] = y[:, :O]
    z_ref[...] = y[:, O:] + b_ref[0]


def _tc_mid_body(agg_ref, cnt_ref, zin_ref, w_ref, b_ref, h_ref, z_ref, *, O, HW):
    cnt = jnp.maximum(cnt_ref[...][:, 0:1], 1.0)
    x = jnp.maximum(agg_ref[...][:, :zin_ref.shape[1]] / cnt + zin_ref[...], 0.0)
    y = jnp.dot(x, w_ref[0], preferred_element_type=jnp.float32)
    h_ref[...] = y if HW == 2 * O else y[:, :O]
    z_ref[...] = y[:, O:] + b_ref[0]


def _tc_final_body(agg_ref, cnt_ref, zin_ref, out_ref):
    cnt = jnp.maximum(cnt_ref[...][:, 0:1], 1.0)
    out_ref[...] = agg_ref[...][:, :zin_ref.shape[1]] / cnt + zin_ref[...]


def _rows_spec(width):
    return pl.BlockSpec((BM, width), lambda i: (i, 0))


def _tc_first(X, W, b, O):
    return pl.pallas_call(
        functools.partial(_tc_first_body, O=O),
        grid=(2 * NP // BM,),
        in_specs=[
            _rows_spec(X.shape[1]),
            pl.BlockSpec((1,) + W.shape[1:], lambda i: (i // NBLK, 0, 0)),
            pl.BlockSpec((1, 1, O), lambda i: (i // NBLK, 0, 0)),
        ],
        out_specs=[_rows_spec(O), _rows_spec(O)],
        out_shape=[jax.ShapeDtypeStruct((2 * NP, O), jnp.float32)] * 2,
    )(X, W, b)


def _tc_mid(AGG, CNT, Z, W, b, O):
    HW = O if O >= 128 else 2 * O
    return pl.pallas_call(
        functools.partial(_tc_mid_body, O=O, HW=HW),
        grid=(2 * NP // BM,),
        in_specs=[
            _rows_spec(AGG.shape[1]),
            _rows_spec(CNT.shape[1]),
            _rows_spec(Z.shape[1]),
            pl.BlockSpec((1,) + W.shape[1:], lambda i: (i // NBLK, 0, 0)),
            pl.BlockSpec((1, 1, O), lambda i: (i // NBLK, 0, 0)),
        ],
        out_specs=[_rows_spec(HW), _rows_spec(O)],
        out_shape=[jax.ShapeDtypeStruct((2 * NP, HW), jnp.float32),
                   jax.ShapeDtypeStruct((2 * NP, O), jnp.float32)],
    )(AGG, CNT, Z, W, b)


def _tc_final(AGG, CNT, Z):
    O = Z.shape[1]
    return pl.pallas_call(
        _tc_final_body,
        grid=(2 * NP // BM,),
        in_specs=[_rows_spec(AGG.shape[1]), _rows_spec(CNT.shape[1]),
                  _rows_spec(O)],
        out_specs=_rows_spec(O),
        out_shape=jax.ShapeDtypeStruct((2 * NP, O), jnp.float32),
    )(AGG, CNT, Z)


def kernel(x_user, x_product, edge_index_rates, edge_index_rated_by, Wl_0_user_rates_product, bl_0_user_rates_product, Wr_0_user_rates_product, etW_0_user_rates_product, etb_0_user_rates_product, Wl_0_product_rated_by_user, bl_0_product_rated_by_user, Wr_0_product_rated_by_user, etW_0_product_rated_by_user, etb_0_product_rated_by_user, Wl_1_user_rates_product, bl_1_user_rates_product, Wr_1_user_rates_product, etW_1_user_rates_product, etb_1_user_rates_product, Wl_1_product_rated_by_user, bl_1_product_rated_by_user, Wr_1_product_rated_by_user, etW_1_product_rated_by_user, etb_1_product_rated_by_user, Wl_2_user_rates_product, bl_2_user_rates_product, Wr_2_user_rates_product, etW_2_user_rates_product, etb_2_user_rates_product, Wl_2_product_rated_by_user, bl_2_product_rated_by_user, Wr_2_product_rated_by_user, etW_2_product_rated_by_user, etb_2_product_rated_by_user):
    prm = dict(locals())

    def fold(l):
        r = "user_rates_product"
        rb = "product_rated_by_user"
        A_r = prm[f"etW_{l}_{r}"] @ prm[f"Wl_{l}_{r}"]
        B_r = prm[f"etW_{l}_{r}"] @ prm[f"Wr_{l}_{r}"]
        c_r = prm[f"etW_{l}_{r}"] @ prm[f"bl_{l}_{r}"] + prm[f"etb_{l}_{r}"]
        A_rb = prm[f"etW_{l}_{rb}"] @ prm[f"Wl_{l}_{rb}"]
        B_rb = prm[f"etW_{l}_{rb}"] @ prm[f"Wr_{l}_{rb}"]
        c_rb = prm[f"etW_{l}_{rb}"] @ prm[f"bl_{l}_{rb}"] + prm[f"etb_{l}_{rb}"]
        W = jnp.stack([
            jnp.concatenate([A_r.T, B_rb.T], axis=1),
            jnp.concatenate([A_rb.T, B_r.T], axis=1),
        ])
        b = jnp.stack([c_rb, c_r])[:, None, :]
        return W, b

    W0, b0 = fold(0)
    W1, b1 = fold(1)
    W2, b2 = fold(2)

    n_pad = EPSC - E
    pad_src = jnp.zeros((n_pad,), jnp.int32)
    pad_dst = jnp.full((n_pad,), NP - 1, jnp.int32)
    src_flat = jnp.concatenate(
        [edge_index_rated_by[0] + NP, pad_src, edge_index_rates[0], pad_src])
    dst_flat = jnp.concatenate(
        [edge_index_rated_by[1], pad_dst, edge_index_rates[1], pad_dst])
    src2 = src_flat.reshape(-1, 1, CHUNK)
    dst2 = dst_flat.reshape(-1, 1, CHUNK)

    zrows = jnp.zeros((NP, 128), jnp.float32)
    ones = jnp.ones((CCNT, 128), jnp.float32)
    pad = jnp.zeros((NP - N, 128), jnp.float32)
    X0 = jnp.concatenate([x_user, pad, x_product, pad])

    CNT = _sc_count()(dst2, zrows, ones)
    H, Z = _tc_first(X0, W0, b0, 128)
    AGG = _sc_segsum()(src2, dst2, H, zrows)
    H, Z = _tc_mid(AGG, CNT, Z, W1, b1, 128)
    AGG = _sc_segsum()(src2, dst2, H, zrows)
    H, Z = _tc_mid(AGG, CNT, Z, W2, b2, 64)
    AGG = _sc_segsum()(src2, dst2, H, zrows)
    OUT = _tc_final(AGG, CNT, Z)
    return OUT[:N], OUT[NP:NP + N]


# speedup vs baseline: 1.6795x; 1.6795x over previous
import functools

import jax
import jax.numpy as jnp
from jax import lax
from jax.experimental import pallas as pl
from jax.experimental.pallas import tpu as pltpu
from jax.experimental.pallas import tpu_sc as plsc

N = 10000
NP = 10240
E = 160000
CHUNK = 128
CCNT = 128
NSUB = 16
CPS = 80
EPSC = NSUB * CPS * CHUNK
RPS = NP // NSUB
BM = 512
NBLK = NP // BM


def _make_sc_segsum():
    scratch = [
        pltpu.VMEM((1, CHUNK), jnp.int32),
        pltpu.VMEM((1, CHUNK), jnp.int32),
        pltpu.VMEM((CHUNK, 128), jnp.float32),
        pltpu.VMEM_SHARED((NP, 128), jnp.float32),
        pltpu.SemaphoreType.DMA,
    ]
    mesh = plsc.VectorSubcoreMesh(core_axis_name="c", subcore_axis_name="s")

    def body(src2, dst2, h, zrows, agg_out, sv, dv, rows, acc, sem):
        c = lax.axis_index("c")
        s = lax.axis_index("s")
        base = s * RPS
        pltpu.sync_copy(zrows.at[pl.ds(0, CHUNK)], rows)
        for k in range(RPS // CHUNK):
            pltpu.sync_copy(rows, acc.at[pl.ds(base + k * CHUNK, CHUNK)])
        plsc.subcore_barrier()

        nreal = EPSC // CHUNK - (EPSC - E) // CHUNK
        lo = (nreal * s) // NSUB
        hi = (nreal * (s + 1)) // NSUB

        def step(q, carry):
            r = c * (NSUB * CPS) + q
            pltpu.sync_copy(src2.at[r], sv)
            pltpu.sync_copy(dst2.at[r], dv)
            pltpu.async_copy(h.at[sv.at[0]], rows, sem).wait()
            pltpu.sync_copy(rows, acc.at[dv.at[0]], add=True)
            return carry

        lax.fori_loop(lo, hi, step, 0)
        plsc.subcore_barrier()

        for k in range(RPS // CHUNK):
            pltpu.sync_copy(acc.at[pl.ds(base + k * CHUNK, CHUNK)], rows)
            pltpu.sync_copy(rows, agg_out.at[pl.ds(c * NP + base + k * CHUNK, CHUNK)])

    return functools.partial(
        pl.kernel, mesh=mesh,
        out_type=jax.ShapeDtypeStruct((2 * NP, 128), jnp.float32),
        scratch_types=scratch,
    )(body)


CCPS = EPSC // CCNT // NSUB


def _make_sc_count():
    scratch = [
        pltpu.VMEM((CCPS, 1, CCNT), jnp.int32),
        pltpu.VMEM((CCNT, 128), jnp.float32),
        pltpu.VMEM_SHARED((NP, 128), jnp.float32),
        pltpu.SemaphoreType.DMA,
    ]
    mesh = plsc.VectorSubcoreMesh(core_axis_name="c", subcore_axis_name="s")

    def body(dst2, zrows, ones, cnt_out, dv, rows, acc, sem_s):
        c = lax.axis_index("c")
        s = lax.axis_index("s")
        base = s * RPS
        pltpu.sync_copy(dst2.at[pl.ds((c * NSUB + s) * CCPS, CCPS)], dv)
        pltpu.sync_copy(zrows.at[pl.ds(0, CCNT)], rows)
        for k in range(RPS // CCNT):
            pltpu.sync_copy(rows, acc.at[pl.ds(base + k * CCNT, CCNT)])
        pltpu.sync_copy(ones, rows)
        plsc.subcore_barrier()

        FIRE = 8

        def step(t, carry):
            for j in range(FIRE):
                pltpu.make_async_copy(rows, acc.at[dv.at[FIRE * t + j, 0]],
                                      sem_s).start(add=True)
            for j in range(FIRE):
                pltpu.make_async_copy(rows, acc.at[pl.ds(0, CCNT)], sem_s).wait()
            return carry

        lax.fori_loop(0, CCPS // FIRE, step, 0)
        plsc.subcore_barrier()

        for k in range(RPS // CCNT):
            pltpu.sync_copy(acc.at[pl.ds(base + k * CCNT, CCNT)], rows)
            pltpu.sync_copy(rows, cnt_out.at[pl.ds(c * NP + base + k * CCNT, CCNT)])

    return functools.partial(
        pl.kernel, mesh=mesh,
        out_type=jax.ShapeDtypeStruct((2 * NP, 128), jnp.float32),
        scratch_types=scratch,
    )(body)


@functools.lru_cache(maxsize=None)
def _sc_segsum():
    return _make_sc_segsum()


@functools.lru_cache(maxsize=None)
def _sc_count():
    return _make_sc_count()


def _tc_first_body(x_ref, w_ref, b_ref, h_ref, z_ref, *, O):
    x = x_ref[...]
    y = jnp.dot(x, w_ref[0], preferred_element_type=jnp.float32)
    h_ref[...] = y[:, :O]
    z_ref[...] = y[:, O:] + b_ref[0]


def _tc_mid_body(agg_ref, cnt_ref, zin_ref, w_ref, b_ref, h_ref, z_ref, *, O, HW):
    cnt = jnp.maximum(cnt_ref[...][:, 0:1], 1.0)
    x = jnp.maximum(agg_ref[...][:, :zin_ref.shape[1]] / cnt + zin_ref[...], 0.0)
    y = jnp.dot(x, w_ref[0], preferred_element_type=jnp.float32)
    h_ref[...] = y if HW == 2 * O else y[:, :O]
    z_ref[...] = y[:, O:] + b_ref[0]


def _tc_final_body(agg_ref, cnt_ref, zin_ref, out_ref):
    cnt = jnp.maximum(cnt_ref[...][:, 0:1], 1.0)
    out_ref[...] = agg_ref[...][:, :zin_ref.shape[1]] / cnt + zin_ref[...]


def _rows_spec(width):
    return pl.BlockSpec((BM, width), lambda i: (i, 0))


def _tc_first(X, W, b, O):
    return pl.pallas_call(
        functools.partial(_tc_first_body, O=O),
        grid=(2 * NP // BM,),
        in_specs=[
            _rows_spec(X.shape[1]),
            pl.BlockSpec((1,) + W.shape[1:], lambda i: (i // NBLK, 0, 0)),
            pl.BlockSpec((1, 1, O), lambda i: (i // NBLK, 0, 0)),
        ],
        out_specs=[_rows_spec(O), _rows_spec(O)],
        out_shape=[jax.ShapeDtypeStruct((2 * NP, O), jnp.float32)] * 2,
    )(X, W, b)


def _tc_mid(AGG, CNT, Z, W, b, O):
    HW = O if O >= 128 else 2 * O
    return pl.pallas_call(
        functools.partial(_tc_mid_body, O=O, HW=HW),
        grid=(2 * NP // BM,),
        in_specs=[
            _rows_spec(AGG.shape[1]),
            _rows_spec(CNT.shape[1]),
            _rows_spec(Z.shape[1]),
            pl.BlockSpec((1,) + W.shape[1:], lambda i: (i // NBLK, 0, 0)),
            pl.BlockSpec((1, 1, O), lambda i: (i // NBLK, 0, 0)),
        ],
        out_specs=[_rows_spec(HW), _rows_spec(O)],
        out_shape=[jax.ShapeDtypeStruct((2 * NP, HW), jnp.float32),
                   jax.ShapeDtypeStruct((2 * NP, O), jnp.float32)],
    )(AGG, CNT, Z, W, b)


def _tc_final(AGG, CNT, Z):
    O = Z.shape[1]
    return pl.pallas_call(
        _tc_final_body,
        grid=(2 * NP // BM,),
        in_specs=[_rows_spec(AGG.shape[1]), _rows_spec(CNT.shape[1]),
                  _rows_spec(O)],
        out_specs=_rows_spec(O),
        out_shape=jax.ShapeDtypeStruct((2 * NP, O), jnp.float32),
    )(AGG, CNT, Z)


def kernel(x_user, x_product, edge_index_rates, edge_index_rated_by, Wl_0_user_rates_product, bl_0_user_rates_product, Wr_0_user_rates_product, etW_0_user_rates_product, etb_0_user_rates_product, Wl_0_product_rated_by_user, bl_0_product_rated_by_user, Wr_0_product_rated_by_user, etW_0_product_rated_by_user, etb_0_product_rated_by_user, Wl_1_user_rates_product, bl_1_user_rates_product, Wr_1_user_rates_product, etW_1_user_rates_product, etb_1_user_rates_product, Wl_1_product_rated_by_user, bl_1_product_rated_by_user, Wr_1_product_rated_by_user, etW_1_product_rated_by_user, etb_1_product_rated_by_user, Wl_2_user_rates_product, bl_2_user_rates_product, Wr_2_user_rates_product, etW_2_user_rates_product, etb_2_user_rates_product, Wl_2_product_rated_by_user, bl_2_product_rated_by_user, Wr_2_product_rated_by_user, etW_2_product_rated_by_user, etb_2_product_rated_by_user):
    prm = dict(locals())

    def fold(l):
        r = "user_rates_product"
        rb = "product_rated_by_user"
        A_r = prm[f"etW_{l}_{r}"] @ prm[f"Wl_{l}_{r}"]
        B_r = prm[f"etW_{l}_{r}"] @ prm[f"Wr_{l}_{r}"]
        c_r = prm[f"etW_{l}_{r}"] @ prm[f"bl_{l}_{r}"] + prm[f"etb_{l}_{r}"]
        A_rb = prm[f"etW_{l}_{rb}"] @ prm[f"Wl_{l}_{rb}"]
        B_rb = prm[f"etW_{l}_{rb}"] @ prm[f"Wr_{l}_{rb}"]
        c_rb = prm[f"etW_{l}_{rb}"] @ prm[f"bl_{l}_{rb}"] + prm[f"etb_{l}_{rb}"]
        W = jnp.stack([
            jnp.concatenate([A_r.T, B_rb.T], axis=1),
            jnp.concatenate([A_rb.T, B_r.T], axis=1),
        ])
        b = jnp.stack([c_rb, c_r])[:, None, :]
        return W, b

    W0, b0 = fold(0)
    W1, b1 = fold(1)
    W2, b2 = fold(2)

    n_pad = EPSC - E
    pad_src = jnp.zeros((n_pad,), jnp.int32)
    pad_dst = jnp.full((n_pad,), NP - 1, jnp.int32)
    src_flat = jnp.concatenate(
        [edge_index_rated_by[0] + NP, pad_src, edge_index_rates[0], pad_src])
    dst_flat = jnp.concatenate(
        [edge_index_rated_by[1], pad_dst, edge_index_rates[1], pad_dst])
    src2 = src_flat.reshape(-1, 1, CHUNK)
    dst2 = dst_flat.reshape(-1, 1, CHUNK)

    zrows = jnp.zeros((NP, 128), jnp.float32)
    ones = jnp.ones((CCNT, 128), jnp.float32)
    pad = jnp.zeros((NP - N, 128), jnp.float32)
    X0 = jnp.concatenate([x_user, pad, x_product, pad])

    CNT = _sc_count()(dst2, zrows, ones)
    H, Z = _tc_first(X0, W0, b0, 128)
    AGG = _sc_segsum()(src2, dst2, H, zrows)
    H, Z = _tc_mid(AGG, CNT, Z, W1, b1, 128)
    AGG = _sc_segsum()(src2, dst2, H, zrows)
    H, Z = _tc_mid(AGG, CNT, Z, W2, b2, 64)
    AGG = _sc_segsum()(src2, dst2, H, zrows)
    OUT = _tc_final(AGG, CNT, Z)
    return OUT[:N], OUT[NP:NP + N]
